# baseline (speedup 1.0000x reference)
import functools

import jax
import jax.numpy as jnp
from jax import lax
from jax.experimental import pallas as pl
from jax.experimental.pallas import tpu as pltpu
from jax.experimental.pallas import tpu_sc as plsc

_K = 1024
_D = 64
_B = 32
_T = 576
_ROWS = _B * _T
_BPS = 16
_NBLK = _B // _BPS

_NW = 32
_LANES = 16


def _argmin_body(xt_ref, cbt_ref, cn_ref, rsq_ref, idx_ref, loss_ref):
    cbt = cbt_ref[...]
    cn = cn_ref[...]
    msum = jnp.zeros((1, 1), jnp.float32)
    parts = []
    for b in range(0, _BPS, 4):
        xp = jnp.concatenate([xt_ref[b + j] for j in range(4)], axis=1)
        rp = jnp.concatenate([rsq_ref[b + j] for j in range(4)], axis=1)
        s2 = lax.dot_general(cbt, xp + xp, (((0,), (0,)), ((), ())),
                             preferred_element_type=jnp.float32)
        d = (rp - s2) + cn
        m = jnp.min(d, axis=0, keepdims=True)
        ii = lax.broadcasted_iota(jnp.int32, d.shape, 0)
        parts.append(jnp.min(jnp.where(d == m, ii, _K), axis=0))
        msum = msum + jnp.sum(m, axis=(0, 1), keepdims=True)
    idx_ref[...] = jnp.concatenate(parts).reshape(_BPS * _T // 128, 128)

    @pl.when(pl.program_id(0) == 0)
    def _():
        loss_ref[...] = jnp.zeros((1, 1), jnp.float32)

    loss_ref[...] += msum

    @pl.when(pl.program_id(0) == _NBLK - 1)
    def _():
        loss_ref[...] = loss_ref[...] * (1.25 / (_ROWS * _D))


_argmin_call = pl.pallas_call(
    _argmin_body,
    grid=(_NBLK,),
    in_specs=[
        pl.BlockSpec((_BPS, _D, _T), lambda i: (i, 0, 0)),
        pl.BlockSpec((_D, _K), lambda i: (0, 0)),
        pl.BlockSpec((_K, 1), lambda i: (0, 0)),
        pl.BlockSpec((_BPS, 1, _T), lambda i: (i, 0, 0)),
    ],
    out_specs=[
        pl.BlockSpec((_BPS * _T // 128, 128), lambda i: (i, 0)),
        pl.BlockSpec((1, 1), lambda i: (0, 0)),
    ],
    out_shape=[
        jax.ShapeDtypeStruct((_ROWS // 128, 128), jnp.int32),
        jax.ShapeDtypeStruct((1, 1), jnp.float32),
    ],
)


@functools.partial(
    pl.kernel,
    mesh=plsc.VectorSubcoreMesh(core_axis_name="c", subcore_axis_name="s"),
    compiler_params=pltpu.CompilerParams(needs_layout_passes=False),
    out_type=jax.ShapeDtypeStruct((_NW, _D, _T), jnp.float32),
    scratch_types=[
        pltpu.VMEM((_D * _K,), jnp.float32),
        pltpu.VMEM((_T,), jnp.int32),
        pltpu.VMEM((_D, _T), jnp.float32),
    ],
)
def _sc_gather_t(cbt_hbm, idx_hbm, out_hbm, cbt_v, idx_v, zqt_v):
    w = lax.axis_index("c") * 16 + lax.axis_index("s")
    pltpu.sync_copy(cbt_hbm, cbt_v)
    pltpu.sync_copy(idx_hbm.at[pl.ds(w * _T, _T)], idx_v)
    nch = _T // _LANES
    idx_chunks = [idx_v[pl.ds(tc * _LANES, _LANES)] for tc in range(nch)]

    @plsc.parallel_loop(0, _D, unroll=8)
    def _(dd):
        off = jnp.full((_LANES,), dd * _K, jnp.int32)
        for tc in range(nch):
            vals = plsc.load_gather(cbt_v, [idx_chunks[tc] + off])
            zqt_v[dd, pl.ds(tc * _LANES, _LANES)] = vals

    pltpu.sync_copy(zqt_v, out_hbm.at[w])


def kernel(x, codebook):
    B, T, D = x.shape
    cbt = jnp.swapaxes(codebook, 0, 1)
    xt = jnp.swapaxes(x, 1, 2)
    cn = jnp.sum(codebook ** 2, axis=1)[:, None]
    rsq = jnp.sum(x ** 2, axis=2)[:, None, :]
    idx2d, loss = _argmin_call(xt, cbt, cn, rsq)
    idx_flat = idx2d.reshape(-1)
    zqt = _sc_gather_t(cbt.reshape(-1), idx_flat)
    zq_st = jnp.swapaxes(zqt, 1, 2)
    return zq_st, loss.reshape(()), idx_flat.reshape(B, T)


# speedup vs baseline: 1.0110x; 1.0110x over previous
import functools

import jax
import jax.numpy as jnp
from jax import lax
from jax.experimental import pallas as pl
from jax.experimental.pallas import tpu as pltpu
from jax.experimental.pallas import tpu_sc as plsc

_K = 1024
_D = 64
_B = 32
_T = 576
_ROWS = _B * _T
_BPS = 16
_NBLK = _B // _BPS

_NW = 32
_LANES = 16


def _argmin_body(xt_ref, cbt_ref, cn_ref, rsq_ref, idx_ref, loss_ref):
    cbt = cbt_ref[...]
    cn = cn_ref[...]
    msum = jnp.zeros((1, 1), jnp.float32)
    parts = []
    for b in range(0, _BPS, 2):
        xp = jnp.concatenate([xt_ref[b], xt_ref[b + 1]], axis=1)
        rp = jnp.concatenate([rsq_ref[b], rsq_ref[b + 1]], axis=1)
        s2 = lax.dot_general(cbt, xp + xp, (((0,), (0,)), ((), ())),
                             preferred_element_type=jnp.float32)
        d = (rp - s2) + cn
        m = jnp.min(d, axis=0, keepdims=True)
        ii = lax.broadcasted_iota(jnp.int32, d.shape, 0)
        parts.append(jnp.min(jnp.where(d == m, ii, _K), axis=0))
        msum = msum + jnp.sum(m, axis=(0, 1), keepdims=True)
    idx_ref[...] = jnp.concatenate(parts).reshape(_BPS * _T // 128, 128)

    @pl.when(pl.program_id(0) == 0)
    def _():
        loss_ref[...] = jnp.zeros((1, 1), jnp.float32)

    loss_ref[...] += msum

    @pl.when(pl.program_id(0) == _NBLK - 1)
    def _():
        loss_ref[...] = loss_ref[...] * (1.25 / (_ROWS * _D))


_argmin_call = pl.pallas_call(
    _argmin_body,
    grid=(_NBLK,),
    in_specs=[
        pl.BlockSpec((_BPS, _D, _T), lambda i: (i, 0, 0)),
        pl.BlockSpec((_D, _K), lambda i: (0, 0)),
        pl.BlockSpec((_K, 1), lambda i: (0, 0)),
        pl.BlockSpec((_BPS, 1, _T), lambda i: (i, 0, 0)),
    ],
    out_specs=[
        pl.BlockSpec((_BPS * _T // 128, 128), lambda i: (i, 0)),
        pl.BlockSpec((1, 1), lambda i: (0, 0)),
    ],
    out_shape=[
        jax.ShapeDtypeStruct((_ROWS // 128, 128), jnp.int32),
        jax.ShapeDtypeStruct((1, 1), jnp.float32),
    ],
)


@functools.partial(
    pl.kernel,
    mesh=plsc.VectorSubcoreMesh(core_axis_name="c", subcore_axis_name="s"),
    compiler_params=pltpu.CompilerParams(needs_layout_passes=False),
    out_type=jax.ShapeDtypeStruct((_NW, _D, _T), jnp.float32),
    scratch_types=[
        pltpu.VMEM((_D * _K,), jnp.float32),
        pltpu.VMEM((_T,), jnp.int32),
        pltpu.VMEM((_D, _T), jnp.float32),
    ],
)
def _sc_gather_t(cbt_hbm, idx_hbm, out_hbm, cbt_v, idx_v, zqt_v):
    w = lax.axis_index("c") * 16 + lax.axis_index("s")
    pltpu.sync_copy(cbt_hbm, cbt_v)
    pltpu.sync_copy(idx_hbm.at[pl.ds(w * _T, _T)], idx_v)
    nch = _T // _LANES
    idx_chunks = [idx_v[pl.ds(tc * _LANES, _LANES)] for tc in range(nch)]

    @plsc.parallel_loop(0, _D, unroll=8)
    def _(dd):
        off = jnp.full((_LANES,), dd * _K, jnp.int32)
        for tc in range(nch):
            vals = plsc.load_gather(cbt_v, [idx_chunks[tc] + off])
            zqt_v[dd, pl.ds(tc * _LANES, _LANES)] = vals

    pltpu.sync_copy(zqt_v, out_hbm.at[w])


def kernel(x, codebook):
    B, T, D = x.shape
    cbt = jnp.swapaxes(codebook, 0, 1)
    xt = jnp.swapaxes(x, 1, 2)
    cn = jnp.sum(codebook ** 2, axis=1)[:, None]
    rsq = jnp.sum(x ** 2, axis=2)[:, None, :]
    idx2d, loss = _argmin_call(xt, cbt, cn, rsq)
    idx_flat = idx2d.reshape(-1)
    zqt = _sc_gather_t(cbt.reshape(-1), idx_flat)
    zq_st = jnp.swapaxes(zqt, 1, 2)
    return zq_st, loss.reshape(()), idx_flat.reshape(B, T)
